# baseline (speedup 1.0000x reference)
import jax
import jax.numpy as jnp
from jax.experimental import pallas as pl

B, S, H = 1, 2048, 1024
E, K, F, KS = 8, 2, 512, 4
N_MLP = 4
TB = 256
T = B * S


def _dense_body(x_ref, xprev_ref, wg_ref, bias_ref, w1_ref, w3_ref, w2_ref,
                cw_ref, cb_ref, out_ref):
    pid = pl.program_id(0)
    x = x_ref[...]

    logits = jax.lax.dot_general(
        x.astype(jnp.bfloat16), wg_ref[...].astype(jnp.bfloat16),
        (((1,), (0,)), ((), ())),
        preferred_element_type=jnp.float32)
    m = jnp.max(logits, axis=-1, keepdims=True)
    unnorm = jnp.exp(logits - m)
    probs = unnorm / jnp.sum(unnorm, axis=-1, keepdims=True)
    sel = probs + bias_ref[...]

    eids = jax.lax.broadcasted_iota(jnp.int32, (TB, E), 1)
    a1 = jnp.argmax(sel, axis=-1)
    oh1 = (eids == a1[:, None])
    sel2 = jnp.where(oh1, -jnp.inf, sel)
    a2 = jnp.argmax(sel2, axis=-1)
    oh2 = (eids == a2[:, None])
    w1g = jnp.sum(jnp.where(oh1, probs, 0.0), axis=-1)
    w2g = jnp.sum(jnp.where(oh2, probs, 0.0), axis=-1)
    denom = w1g + w2g + 1e-9
    comb = (jnp.where(oh1, (w1g / denom)[:, None], 0.0)
            + jnp.where(oh2, (w2g / denom)[:, None], 0.0))

    acc = jnp.zeros((TB, H), jnp.float32)
    for e in range(N_MLP):
        a = jax.lax.dot_general(x, w1_ref[e], (((1,), (0,)), ((), ())),
                                preferred_element_type=jnp.float32)
        b = jax.lax.dot_general(x, w3_ref[e], (((1,), (0,)), ((), ())),
                                preferred_element_type=jnp.float32)
        h = (a * jax.nn.sigmoid(a)) * b
        y = jax.lax.dot_general(h, w2_ref[e], (((1,), (0,)), ((), ())),
                                preferred_element_type=jnp.float32)
        acc = acc + comb[:, e:e + 1] * y

    acc = acc + (comb[:, 4:5] + comb[:, 6:7]) * x
    acc = acc + comb[:, 7:8] * jnp.maximum(x, 0.0)

    halo = xprev_ref[TB - (KS - 1):TB, :]
    halo = jnp.where(pid == 0, 0.0, halo)
    xh = jnp.concatenate([halo, x], axis=0)
    shared = cb_ref[...]
    for j in range(KS):
        shared = shared + xh[j:j + TB, :] * cw_ref[j][None, :]

    out_ref[...] = acc + shared


@jax.jit
def kernel(hidden_states, Wg, gate_bias, W1, W3, W2, conv_w, conv_b):
    flat = hidden_states.reshape(T, H)
    cw = conv_w.T
    cb = conv_b.reshape(1, H)
    bias = gate_bias.reshape(1, E)

    grid = (T // TB,)
    out = pl.pallas_call(
        _dense_body,
        grid=grid,
        in_specs=[
            pl.BlockSpec((TB, H), lambda i: (i, 0)),
            pl.BlockSpec((TB, H), lambda i: (jnp.maximum(i - 1, 0), 0)),
            pl.BlockSpec((H, E), lambda i: (0, 0)),
            pl.BlockSpec((1, E), lambda i: (0, 0)),
            pl.BlockSpec((N_MLP, H, F), lambda i: (0, 0, 0)),
            pl.BlockSpec((N_MLP, H, F), lambda i: (0, 0, 0)),
            pl.BlockSpec((N_MLP, F, H), lambda i: (0, 0, 0)),
            pl.BlockSpec((KS, H), lambda i: (0, 0)),
            pl.BlockSpec((1, H), lambda i: (0, 0)),
        ],
        out_specs=pl.BlockSpec((TB, H), lambda i: (i, 0)),
        out_shape=jax.ShapeDtypeStruct((T, H), jnp.float32),
    )(flat, flat, Wg, bias, W1, W3, W2, cw, cb)
    return out.reshape(B, S, H)


# speedup vs baseline: 1.0028x; 1.0028x over previous
import jax
import jax.numpy as jnp
from jax.experimental import pallas as pl
from jax.experimental.pallas import tpu as pltpu

B, S, H = 1, 2048, 1024
E, K, F, KS = 8, 2, 512, 4
N_MLP = 4
TB = 256
T = B * S


def _dense_body(x_ref, wg_ref, bias_ref, w1_ref, w3_ref, w2_ref,
                cw_ref, cb_ref, out_ref, halo_ref):
    pid = pl.program_id(0)
    x = x_ref[...]

    logits = jax.lax.dot_general(
        x.astype(jnp.bfloat16), wg_ref[...].astype(jnp.bfloat16),
        (((1,), (0,)), ((), ())),
        preferred_element_type=jnp.float32)
    m = jnp.max(logits, axis=-1, keepdims=True)
    unnorm = jnp.exp(logits - m)
    probs = unnorm / jnp.sum(unnorm, axis=-1, keepdims=True)
    sel = probs + bias_ref[...]

    eids = jax.lax.broadcasted_iota(jnp.int32, (TB, E), 1)
    a1 = jnp.argmax(sel, axis=-1)
    oh1 = (eids == a1[:, None])
    sel2 = jnp.where(oh1, -jnp.inf, sel)
    a2 = jnp.argmax(sel2, axis=-1)
    oh2 = (eids == a2[:, None])
    w1g = jnp.sum(jnp.where(oh1, probs, 0.0), axis=-1)
    w2g = jnp.sum(jnp.where(oh2, probs, 0.0), axis=-1)
    denom = w1g + w2g + 1e-9
    comb = (jnp.where(oh1, (w1g / denom)[:, None], 0.0)
            + jnp.where(oh2, (w2g / denom)[:, None], 0.0))

    acc = jnp.zeros((TB, H), jnp.float32)
    for e in range(N_MLP):
        a = jax.lax.dot_general(x, w1_ref[e], (((1,), (0,)), ((), ())),
                                preferred_element_type=jnp.float32)
        b = jax.lax.dot_general(x, w3_ref[e], (((1,), (0,)), ((), ())),
                                preferred_element_type=jnp.float32)
        h = (a * jax.nn.sigmoid(a)) * b
        y = jax.lax.dot_general(h, w2_ref[e], (((1,), (0,)), ((), ())),
                                preferred_element_type=jnp.float32)
        acc = acc + comb[:, e:e + 1] * y

    acc = acc + (comb[:, 4:5] + comb[:, 6:7]) * x
    acc = acc + comb[:, 7:8] * jnp.maximum(x, 0.0)

    halo = jnp.where(pid == 0, 0.0, halo_ref[...])
    xh = jnp.concatenate([halo, x], axis=0)
    shared = cb_ref[...]
    for j in range(KS):
        shared = shared + xh[j:j + TB, :] * cw_ref[j][None, :]

    halo_ref[...] = x[TB - (KS - 1):TB, :]
    out_ref[...] = acc + shared


@jax.jit
def kernel(hidden_states, Wg, gate_bias, W1, W3, W2, conv_w, conv_b):
    flat = hidden_states.reshape(T, H)
    cw = conv_w.T
    cb = conv_b.reshape(1, H)
    bias = gate_bias.reshape(1, E)

    grid = (T // TB,)
    out = pl.pallas_call(
        _dense_body,
        grid=grid,
        in_specs=[
            pl.BlockSpec((TB, H), lambda i: (i, 0)),
            pl.BlockSpec((H, E), lambda i: (0, 0)),
            pl.BlockSpec((1, E), lambda i: (0, 0)),
            pl.BlockSpec((N_MLP, H, F), lambda i: (0, 0, 0)),
            pl.BlockSpec((N_MLP, H, F), lambda i: (0, 0, 0)),
            pl.BlockSpec((N_MLP, F, H), lambda i: (0, 0, 0)),
            pl.BlockSpec((KS, H), lambda i: (0, 0)),
            pl.BlockSpec((1, H), lambda i: (0, 0)),
        ],
        out_specs=pl.BlockSpec((TB, H), lambda i: (i, 0)),
        out_shape=jax.ShapeDtypeStruct((T, H), jnp.float32),
        scratch_shapes=[pltpu.VMEM((KS - 1, H), jnp.float32)],
    )(flat, Wg, bias, W1, W3, W2, cw, cb)
    return out.reshape(B, S, H)


# speedup vs baseline: 1.0510x; 1.0481x over previous
import jax
import jax.numpy as jnp
from jax.experimental import pallas as pl
from jax.experimental.pallas import tpu as pltpu

B, S, H = 1, 2048, 1024
E, K, F, KS = 8, 2, 512, 4
N_MLP = 4
TB = 512
T = B * S


def _dense_body(x_ref, wg_ref, bias_ref, w1_ref, w3_ref, w2_ref,
                cw_ref, cb_ref, out_ref, halo_ref):
    pid = pl.program_id(0)
    x = x_ref[...]

    logits = jax.lax.dot_general(
        x.astype(jnp.bfloat16), wg_ref[...].astype(jnp.bfloat16),
        (((1,), (0,)), ((), ())),
        preferred_element_type=jnp.float32)
    m = jnp.max(logits, axis=-1, keepdims=True)
    unnorm = jnp.exp(logits - m)
    probs = unnorm / jnp.sum(unnorm, axis=-1, keepdims=True)
    sel = probs + bias_ref[...]

    eids = jax.lax.broadcasted_iota(jnp.int32, (TB, E), 1)
    a1 = jnp.argmax(sel, axis=-1)
    oh1 = (eids == a1[:, None])
    sel2 = jnp.where(oh1, -jnp.inf, sel)
    a2 = jnp.argmax(sel2, axis=-1)
    oh2 = (eids == a2[:, None])
    w1g = jnp.sum(jnp.where(oh1, probs, 0.0), axis=-1)
    w2g = jnp.sum(jnp.where(oh2, probs, 0.0), axis=-1)
    denom = w1g + w2g + 1e-9
    comb = (jnp.where(oh1, (w1g / denom)[:, None], 0.0)
            + jnp.where(oh2, (w2g / denom)[:, None], 0.0))

    acc = jnp.zeros((TB, H), jnp.float32)
    for e in range(N_MLP):
        a = jax.lax.dot_general(x, w1_ref[e], (((1,), (0,)), ((), ())),
                                preferred_element_type=jnp.float32)
        b = jax.lax.dot_general(x, w3_ref[e], (((1,), (0,)), ((), ())),
                                preferred_element_type=jnp.float32)
        h = (a * jax.nn.sigmoid(a)) * b
        y = jax.lax.dot_general(h, w2_ref[e], (((1,), (0,)), ((), ())),
                                preferred_element_type=jnp.float32)
        acc = acc + comb[:, e:e + 1] * y

    acc = acc + (comb[:, 4:5] + comb[:, 6:7]) * x
    acc = acc + comb[:, 7:8] * jnp.maximum(x, 0.0)

    halo = jnp.where(pid == 0, 0.0, halo_ref[...])
    xh = jnp.concatenate([halo, x], axis=0)
    shared = cb_ref[...]
    for j in range(KS):
        shared = shared + xh[j:j + TB, :] * cw_ref[j][None, :]

    halo_ref[...] = x[TB - (KS - 1):TB, :]
    out_ref[...] = acc + shared


@jax.jit
def kernel(hidden_states, Wg, gate_bias, W1, W3, W2, conv_w, conv_b):
    flat = hidden_states.reshape(T, H)
    cw = conv_w.T
    cb = conv_b.reshape(1, H)
    bias = gate_bias.reshape(1, E)

    grid = (T // TB,)
    out = pl.pallas_call(
        _dense_body,
        grid=grid,
        in_specs=[
            pl.BlockSpec((TB, H), lambda i: (i, 0)),
            pl.BlockSpec((H, E), lambda i: (0, 0)),
            pl.BlockSpec((1, E), lambda i: (0, 0)),
            pl.BlockSpec((N_MLP, H, F), lambda i: (0, 0, 0)),
            pl.BlockSpec((N_MLP, H, F), lambda i: (0, 0, 0)),
            pl.BlockSpec((N_MLP, F, H), lambda i: (0, 0, 0)),
            pl.BlockSpec((KS, H), lambda i: (0, 0)),
            pl.BlockSpec((1, H), lambda i: (0, 0)),
        ],
        out_specs=pl.BlockSpec((TB, H), lambda i: (i, 0)),
        out_shape=jax.ShapeDtypeStruct((T, H), jnp.float32),
        scratch_shapes=[pltpu.VMEM((KS - 1, H), jnp.float32)],
    )(flat, Wg, bias, W1, W3, W2, cw, cb)
    return out.reshape(B, S, H)
